# baseline (speedup 1.0000x reference)
import functools

import jax
import jax.numpy as jnp
from jax import lax
from jax.experimental import pallas as pl
from jax.experimental.pallas import tpu as pltpu
from jax.experimental.pallas import tpu_sc as plsc

VOCAB = 1000000
EMBED = 64
CTX = 200
BATCH = 1024

NUM_WORKERS = 32
SROWS = VOCAB // 2
FULL_BLOCKS = VOCAB // 128
TAIL_ROWS = VOCAB - FULL_BLOCKS * 128
BASE_BLOCKS = FULL_BLOCKS // NUM_WORKERS
EXTRA_WORKERS = FULL_BLOCKS - BASE_BLOCKS * NUM_WORKERS
UNITS = CTX * (BATCH // 128)
UNITS_PER_W = UNITS // NUM_WORKERS


def _iota16():
    return lax.iota(jnp.int32, 16)


def _splat(v):
    return jnp.full((16,), v, jnp.int32)


def _transpose_block(src, dst, n_srows, cvecs):
    for s in range(n_srows):
        for j8 in range(8):
            rl = 2 * s + (1 if j8 >= 4 else 0)
            v = plsc.load_gather(src, [cvecs[j8 % 4], _splat(rl)])
            dst[s, pl.ds(16 * j8, 16)] = v


def _conv_body(tt_hbm, tail_hbm, t2s_hbm,
               ibuf, tbuf, tailv, isem0, isem1, osem0, osem1):
    cidx = lax.axis_index("c")
    sidx = lax.axis_index("s")
    wid = sidx * 2 + cidx
    start = wid * BASE_BLOCKS + jnp.minimum(wid, EXTRA_WORKERS)

    iot = _iota16()
    cvecs = [iot + 16 * k for k in range(4)]
    isems = (isem0, isem1)
    osems = (osem0, osem1)

    def in_start(i, b):
        j = start + i
        pltpu.async_copy(tt_hbm.at[:, pl.ds(j * 128, 128)], ibuf.at[b],
                         isems[b])

    def in_wait(i, b):
        j = start + i
        pltpu.make_async_copy(tt_hbm.at[:, pl.ds(j * 128, 128)], ibuf.at[b],
                              isems[b]).wait()

    def out_start(i, b):
        j = start + i
        pltpu.async_copy(tbuf.at[b], t2s_hbm.at[pl.ds(j * 64, 64)], osems[b])

    def out_wait(i, b):
        j = start + i
        pltpu.make_async_copy(tbuf.at[b], t2s_hbm.at[pl.ds(j * 64, 64)],
                              osems[b]).wait()

    in_start(0, 0)

    def outer(o, _):
        for b in range(2):
            i = 2 * o + b
            nxt = 1 - b
            pl.when(i + 1 < BASE_BLOCKS)(lambda: in_start(i + 1, nxt))
            in_wait(i, b)
            pl.when(i >= 2)(lambda: out_wait(i - 2, b))
            _transpose_block(ibuf.at[b], tbuf.at[b], 64, cvecs)
            out_start(i, b)
        return 0

    lax.fori_loop(0, BASE_BLOCKS // 2, outer, 0)
    out_wait(BASE_BLOCKS - 2, 0)
    out_wait(BASE_BLOCKS - 1, 1)

    @pl.when(wid < EXTRA_WORKERS)
    def _extra():
        j = start + BASE_BLOCKS
        pltpu.sync_copy(tt_hbm.at[:, pl.ds(j * 128, 128)], ibuf.at[0])
        _transpose_block(ibuf.at[0], tbuf.at[0], 64, cvecs)
        pltpu.sync_copy(tbuf.at[0], t2s_hbm.at[pl.ds(j * 64, 64)])

    @pl.when(wid == NUM_WORKERS - 1)
    def _tail():
        pltpu.sync_copy(tail_hbm, tailv)
        for s in range(TAIL_ROWS // 2):
            for j8 in range(8):
                rl = 2 * s + (1 if j8 >= 4 else 0)
                idxv = cvecs[j8 % 4] * 64 + _splat(rl)
                tbuf[0, s, pl.ds(16 * j8, 16)] = plsc.load_gather(
                    tailv, [idxv])
        pltpu.sync_copy(tbuf.at[0, pl.ds(0, TAIL_ROWS // 2)],
                        t2s_hbm.at[pl.ds(FULL_BLOCKS * 64, TAIL_ROWS // 2)])


def _gather_body(xt_hbm, posf_hbm, t2s_hbm, out_hbm,
                 idx_v, sv, rows_v, otile_v, posf_v,
                 isem0, isem1, rsem0, rsem1, wsem0, wsem1):
    cidx = lax.axis_index("c")
    sidx = lax.axis_index("s")
    wid = sidx * 2 + cidx
    base = wid * UNITS_PER_W

    iot = _iota16()
    isems = (isem0, isem1)
    rsems = (rsem0, rsem1)
    wsems = (wsem0, wsem1)

    pltpu.sync_copy(posf_hbm, posf_v)

    def unit_pb(i):
        uid = base + i
        return uid // 8, uid % 8

    def idx_start(i, b):
        p, bb = unit_pb(i)
        pltpu.async_copy(xt_hbm.at[p, bb], idx_v.at[b], isems[b])

    def idx_wait(i, b):
        p, bb = unit_pb(i)
        pltpu.make_async_copy(xt_hbm.at[p, bb], idx_v.at[b],
                              isems[b]).wait()

    def rows_start(b):
        pltpu.async_copy(t2s_hbm.at[sv.at[b]], rows_v.at[b], rsems[b])

    def rows_wait(b):
        pltpu.make_async_copy(t2s_hbm.at[sv.at[b]], rows_v.at[b],
                              rsems[b]).wait()

    def sv_compute(b):
        for g in range(8):
            sl = pl.ds(g * 16, 16)
            sv[b, sl] = lax.shift_right_logical(idx_v[b, sl], 1)

    def out_start(i, b):
        p, bb = unit_pb(i)
        for cb in range(8):
            pltpu.async_copy(otile_v.at[b, pl.ds(cb * 8, 8)],
                             out_hbm.at[p, cb, bb], wsems[b])

    def out_wait(i, b):
        p, bb = unit_pb(i)
        for cb in range(8):
            pltpu.make_async_copy(otile_v.at[b, pl.ds(cb * 8, 8)],
                                  out_hbm.at[p, cb, bb], wsems[b]).wait()

    def compute(i, b):
        p, _ = unit_pb(i)
        hg64 = []
        rowg = []
        for g in range(8):
            idxg = idx_v[b, pl.ds(g * 16, 16)]
            hg64.append(lax.shift_left(lax.bitwise_and(idxg, 1), 6))
            rowg.append(iot + g * 16)

        def cf_body(cf, _):
            for cc in range(8):
                c = cf * 8 + cc
                pidx = _splat(c * 200 + p)
                psp = plsc.load_gather(posf_v, [pidx])
                csp = _splat(c)
                for g in range(8):
                    colv = hg64[g] + csp
                    v = plsc.load_gather(rows_v.at[b], [rowg[g], colv])
                    otile_v[b, c, pl.ds(g * 16, 16)] = v + psp
            return 0

        lax.fori_loop(0, 8, cf_body, 0)

    pltpu.sync_copy(xt_hbm.at[base // 8, base % 8], idx_v.at[0])
    sv_compute(0)
    rows_start(0)

    def _next_gather(i, nxt):
        idx_wait(i + 1, nxt)
        sv_compute(nxt)
        rows_start(nxt)

    def outer(o, _):
        for b in range(2):
            i = 2 * o + b
            nxt = 1 - b
            if b == 0:
                idx_start(i + 1, nxt)
            else:
                pl.when(o < UNITS_PER_W // 2 - 1)(
                    lambda: idx_start(i + 1, nxt))
            rows_wait(b)
            pl.when(i >= 2)(lambda: out_wait(i - 2, b))
            compute(i, b)
            out_start(i, b)
            if b == 0:
                _next_gather(i, nxt)
            else:
                pl.when(o < UNITS_PER_W // 2 - 1)(
                    lambda: _next_gather(i, nxt))
        return 0

    lax.fori_loop(0, UNITS_PER_W // 2, outer, 0)
    out_wait(UNITS_PER_W - 2, 0)
    out_wait(UNITS_PER_W - 1, 1)


@jax.jit
def _emb_call(x, table, pos_enc):
    mesh = plsc.VectorSubcoreMesh(core_axis_name="c", subcore_axis_name="s")

    tt = table.T
    tail = tt[:, FULL_BLOCKS * 128:].reshape(64 * TAIL_ROWS)
    xt3 = x.astype(jnp.int32).T.reshape(CTX, 8, 128)
    posf = pos_enc.T.reshape(EMBED * CTX)

    conv = functools.partial(
        pl.kernel,
        mesh=mesh,
        out_type=jax.ShapeDtypeStruct((SROWS, 128), jnp.float32),
        scratch_types=[
            pltpu.VMEM((2, 64, 128), jnp.float32),
            pltpu.VMEM((2, 64, 128), jnp.float32),
            pltpu.VMEM((64 * TAIL_ROWS,), jnp.float32),
        ] + [pltpu.SemaphoreType.DMA] * 4,
        compiler_params=pltpu.CompilerParams(needs_layout_passes=False),
    )(_conv_body)
    t2s = conv(tt, tail)

    gath = functools.partial(
        pl.kernel,
        mesh=mesh,
        out_type=jax.ShapeDtypeStruct((CTX, 8, 8, 8, 128), jnp.float32),
        scratch_types=[
            pltpu.VMEM((2, 128), jnp.int32),
            pltpu.VMEM((2, 128), jnp.int32),
            pltpu.VMEM((2, 128, 128), jnp.float32),
            pltpu.VMEM((2, 64, 128), jnp.float32),
            pltpu.VMEM((EMBED * CTX,), jnp.float32),
        ] + [pltpu.SemaphoreType.DMA] * 6,
        compiler_params=pltpu.CompilerParams(needs_layout_passes=False),
    )(_gather_body)
    out5 = gath(xt3, posf, t2s)

    return out5.transpose((2, 4, 0, 1, 3)).reshape(BATCH, CTX, EMBED)


def kernel(x, table, pos_enc):
    return _emb_call(x, table, pos_enc)


# speedup vs baseline: 1.4113x; 1.4113x over previous
import functools

import jax
import jax.numpy as jnp
from jax import lax
from jax.experimental import pallas as pl
from jax.experimental.pallas import tpu as pltpu
from jax.experimental.pallas import tpu_sc as plsc

VOCAB = 1000000
EMBED = 64
CTX = 200
BATCH = 1024

NUM_WORKERS = 32
SROWS = VOCAB // 2
FULL_BLOCKS = VOCAB // 128
TAIL_ROWS = VOCAB - FULL_BLOCKS * 128
BASE_BLOCKS = FULL_BLOCKS // NUM_WORKERS
EXTRA_WORKERS = FULL_BLOCKS - BASE_BLOCKS * NUM_WORKERS
UNITS = CTX * (BATCH // 128)
UNITS_PER_W = UNITS // NUM_WORKERS


def _iota16():
    return lax.iota(jnp.int32, 16)


def _splat(v):
    return jnp.full((16,), v, jnp.int32)


def _transpose_block(src, dst, n_srows, cvecs):
    for s0 in range(0, n_srows, 2):
        vs = []
        for s in (s0, s0 + 1):
            for j8 in range(8):
                rl = 2 * s + (1 if j8 >= 4 else 0)
                vs.append(plsc.load_gather(src, [cvecs[j8 % 4], _splat(rl)]))
        k = 0
        for s in (s0, s0 + 1):
            for j8 in range(8):
                dst[s, pl.ds(16 * j8, 16)] = vs[k]
                k += 1


def _conv_body(tt_hbm, tail_hbm, t2s_hbm,
               ibuf, tbuf, tailv, isem0, isem1, osem0, osem1):
    cidx = lax.axis_index("c")
    sidx = lax.axis_index("s")
    wid = sidx * 2 + cidx
    start = wid * BASE_BLOCKS + jnp.minimum(wid, EXTRA_WORKERS)

    iot = _iota16()
    cvecs = [iot + 16 * k for k in range(4)]
    isems = (isem0, isem1)
    osems = (osem0, osem1)

    def in_start(i, b):
        j = start + i
        pltpu.async_copy(tt_hbm.at[:, pl.ds(j * 128, 128)], ibuf.at[b],
                         isems[b])

    def in_wait(i, b):
        j = start + i
        pltpu.make_async_copy(tt_hbm.at[:, pl.ds(j * 128, 128)], ibuf.at[b],
                              isems[b]).wait()

    def out_start(i, b):
        j = start + i
        pltpu.async_copy(tbuf.at[b], t2s_hbm.at[pl.ds(j * 64, 64)], osems[b])

    def out_wait(i, b):
        j = start + i
        pltpu.make_async_copy(tbuf.at[b], t2s_hbm.at[pl.ds(j * 64, 64)],
                              osems[b]).wait()

    in_start(0, 0)

    def outer(o, _):
        for b in range(2):
            i = 2 * o + b
            nxt = 1 - b
            pl.when(i + 1 < BASE_BLOCKS)(lambda: in_start(i + 1, nxt))
            in_wait(i, b)
            pl.when(i >= 2)(lambda: out_wait(i - 2, b))
            _transpose_block(ibuf.at[b], tbuf.at[b], 64, cvecs)
            out_start(i, b)
        return 0

    lax.fori_loop(0, BASE_BLOCKS // 2, outer, 0)
    out_wait(BASE_BLOCKS - 2, 0)
    out_wait(BASE_BLOCKS - 1, 1)

    @pl.when(wid < EXTRA_WORKERS)
    def _extra():
        j = start + BASE_BLOCKS
        pltpu.sync_copy(tt_hbm.at[:, pl.ds(j * 128, 128)], ibuf.at[0])
        _transpose_block(ibuf.at[0], tbuf.at[0], 64, cvecs)
        pltpu.sync_copy(tbuf.at[0], t2s_hbm.at[pl.ds(j * 64, 64)])

    @pl.when(wid == NUM_WORKERS - 1)
    def _tail():
        pltpu.sync_copy(tail_hbm, tailv)
        for s0 in range(0, TAIL_ROWS // 2, 2):
            vs = []
            for s in (s0, s0 + 1):
                for j8 in range(8):
                    rl = 2 * s + (1 if j8 >= 4 else 0)
                    idxv = cvecs[j8 % 4] * 64 + _splat(rl)
                    vs.append(plsc.load_gather(tailv, [idxv]))
            k = 0
            for s in (s0, s0 + 1):
                for j8 in range(8):
                    tbuf[0, s, pl.ds(16 * j8, 16)] = vs[k]
                    k += 1
        pltpu.sync_copy(tbuf.at[0, pl.ds(0, TAIL_ROWS // 2)],
                        t2s_hbm.at[pl.ds(FULL_BLOCKS * 64, TAIL_ROWS // 2)])


def _gather_body(xt_hbm, posf_hbm, t2s_hbm, out_hbm,
                 idx_v, sv, rows_v, otile_v, posf_v,
                 isem0, isem1, rsem0, rsem1, wsem0, wsem1):
    cidx = lax.axis_index("c")
    sidx = lax.axis_index("s")
    wid = sidx * 2 + cidx
    base = wid * UNITS_PER_W

    iot = _iota16()
    isems = (isem0, isem1)
    rsems = (rsem0, rsem1)
    wsems = (wsem0, wsem1)

    pltpu.sync_copy(posf_hbm, posf_v)

    def unit_pb(i):
        uid = base + i
        return uid // 8, uid % 8

    def idx_start(i, b):
        p, bb = unit_pb(i)
        pltpu.async_copy(xt_hbm.at[p, bb], idx_v.at[b], isems[b])

    def idx_wait(i, b):
        p, bb = unit_pb(i)
        pltpu.make_async_copy(xt_hbm.at[p, bb], idx_v.at[b],
                              isems[b]).wait()

    def rows_start(b):
        pltpu.async_copy(t2s_hbm.at[sv.at[b]], rows_v.at[b], rsems[b])

    def rows_wait(b):
        pltpu.make_async_copy(t2s_hbm.at[sv.at[b]], rows_v.at[b],
                              rsems[b]).wait()

    def sv_compute(b):
        for g in range(8):
            sl = pl.ds(g * 16, 16)
            sv[b, sl] = lax.shift_right_logical(idx_v[b, sl], 1)

    def out_start(i, b):
        p, bb = unit_pb(i)
        for cb in range(8):
            pltpu.async_copy(otile_v.at[b, pl.ds(cb * 8, 8)],
                             out_hbm.at[p, cb, bb], wsems[b])

    def out_wait(i, b):
        p, bb = unit_pb(i)
        for cb in range(8):
            pltpu.make_async_copy(otile_v.at[b, pl.ds(cb * 8, 8)],
                                  out_hbm.at[p, cb, bb], wsems[b]).wait()

    def compute(i, b):
        p, _ = unit_pb(i)
        hg64 = []
        rowg = []
        for g in range(8):
            idxg = idx_v[b, pl.ds(g * 16, 16)]
            hg64.append(lax.shift_left(lax.bitwise_and(idxg, 1), 6))
            rowg.append(iot + g * 16)

        def cf_body(cf, _):
            for cc in range(8):
                c = cf * 8 + cc
                psp = plsc.load_gather(posf_v, [_splat(c * 200 + p)])
                csp = _splat(c)
                colvs = [hg64[g] + csp for g in range(8)]
                vs = [plsc.load_gather(rows_v.at[b], [rowg[g], colvs[g]])
                      for g in range(8)]
                for g in range(8):
                    otile_v[b, c, pl.ds(g * 16, 16)] = vs[g] + psp
            return 0

        lax.fori_loop(0, 8, cf_body, 0)

    pltpu.sync_copy(xt_hbm.at[base // 8, base % 8], idx_v.at[0])
    sv_compute(0)
    rows_start(0)

    def _next_gather(i, nxt):
        idx_wait(i + 1, nxt)
        sv_compute(nxt)
        rows_start(nxt)

    def outer(o, _):
        for b in range(2):
            i = 2 * o + b
            nxt = 1 - b
            if b == 0:
                idx_start(i + 1, nxt)
                rows_wait(b)
                _next_gather(i, nxt)
            else:
                pl.when(o < UNITS_PER_W // 2 - 1)(
                    lambda: idx_start(i + 1, nxt))
                rows_wait(b)
                pl.when(o < UNITS_PER_W // 2 - 1)(
                    lambda: _next_gather(i, nxt))
            pl.when(i >= 2)(lambda: out_wait(i - 2, b))
            compute(i, b)
            out_start(i, b)
        return 0

    lax.fori_loop(0, UNITS_PER_W // 2, outer, 0)
    out_wait(UNITS_PER_W - 2, 0)
    out_wait(UNITS_PER_W - 1, 1)


@jax.jit
def _emb_call(x, table, pos_enc):
    mesh = plsc.VectorSubcoreMesh(core_axis_name="c", subcore_axis_name="s")

    tt = table.T
    tail = tt[:, FULL_BLOCKS * 128:].reshape(64 * TAIL_ROWS)
    xt3 = x.astype(jnp.int32).T.reshape(CTX, 8, 128)
    posf = pos_enc.T.reshape(EMBED * CTX)

    conv = functools.partial(
        pl.kernel,
        mesh=mesh,
        out_type=jax.ShapeDtypeStruct((SROWS, 128), jnp.float32),
        scratch_types=[
            pltpu.VMEM((2, 64, 128), jnp.float32),
            pltpu.VMEM((2, 64, 128), jnp.float32),
            pltpu.VMEM((64 * TAIL_ROWS,), jnp.float32),
        ] + [pltpu.SemaphoreType.DMA] * 4,
        compiler_params=pltpu.CompilerParams(needs_layout_passes=False),
    )(_conv_body)
    t2s = conv(tt, tail)

    gath = functools.partial(
        pl.kernel,
        mesh=mesh,
        out_type=jax.ShapeDtypeStruct((CTX, 8, 8, 8, 128), jnp.float32),
        scratch_types=[
            pltpu.VMEM((2, 128), jnp.int32),
            pltpu.VMEM((2, 128), jnp.int32),
            pltpu.VMEM((2, 128, 128), jnp.float32),
            pltpu.VMEM((2, 64, 128), jnp.float32),
            pltpu.VMEM((EMBED * CTX,), jnp.float32),
        ] + [pltpu.SemaphoreType.DMA] * 6,
        compiler_params=pltpu.CompilerParams(needs_layout_passes=False),
    )(_gather_body)
    out5 = gath(xt3, posf, t2s)

    return out5.transpose((2, 4, 0, 1, 3)).reshape(BATCH, CTX, EMBED)


def kernel(x, table, pos_enc):
    return _emb_call(x, table, pos_enc)


# speedup vs baseline: 2.0047x; 1.4205x over previous
import functools

import jax
import jax.numpy as jnp
from jax import lax
from jax.experimental import pallas as pl
from jax.experimental.pallas import tpu as pltpu
from jax.experimental.pallas import tpu_sc as plsc

VOCAB = 1000000
EMBED = 64
CTX = 200
BATCH = 1024

NUM_WORKERS = 32
SROWS = VOCAB // 2
FULL_BLOCKS = VOCAB // 128
TAIL_ROWS = VOCAB - FULL_BLOCKS * 128
BASE_BLOCKS = FULL_BLOCKS // NUM_WORKERS
EXTRA_WORKERS = FULL_BLOCKS - BASE_BLOCKS * NUM_WORKERS
UNITS = CTX * (BATCH // 128)
UNITS_PER_W = UNITS // NUM_WORKERS


def _iota16():
    return lax.iota(jnp.int32, 16)


def _splat(v):
    return jnp.full((16,), v, jnp.int32)


def _transpose_block(src, dst, io, h64v, shalf):
    def rb_body(rB, _):
        rlv = io + rB * 16
        srowv = shalf + rB * 8
        for cB in range(4):
            scbase = h64v + cB * 16
            for k in range(16):
                dk = lax.bitwise_and(io + k, 15)
                cdiag = dk + cB * 16
                v = plsc.load_gather(src, [cdiag, rlv])
                plsc.store_scatter(dst, [srowv, scbase + dk], v)
        return 0

    lax.fori_loop(0, 8, rb_body, 0)


def _conv_body(tt_hbm, tail_hbm, t2s_hbm,
               ibuf, tbuf, tailv, isem0, isem1, osem0, osem1):
    cidx = lax.axis_index("c")
    sidx = lax.axis_index("s")
    wid = sidx * 2 + cidx
    start = wid * BASE_BLOCKS + jnp.minimum(wid, EXTRA_WORKERS)

    iot = _iota16()
    cvecs = [iot + 16 * k for k in range(4)]
    h64v = lax.shift_left(lax.bitwise_and(iot, 1), 6)
    shalf = lax.shift_right_logical(iot, 1)
    isems = (isem0, isem1)
    osems = (osem0, osem1)

    def in_start(i, b):
        j = start + i
        pltpu.async_copy(tt_hbm.at[:, pl.ds(j * 128, 128)], ibuf.at[b],
                         isems[b])

    def in_wait(i, b):
        j = start + i
        pltpu.make_async_copy(tt_hbm.at[:, pl.ds(j * 128, 128)], ibuf.at[b],
                              isems[b]).wait()

    def out_start(i, b):
        j = start + i
        pltpu.async_copy(tbuf.at[b], t2s_hbm.at[pl.ds(j * 64, 64)], osems[b])

    def out_wait(i, b):
        j = start + i
        pltpu.make_async_copy(tbuf.at[b], t2s_hbm.at[pl.ds(j * 64, 64)],
                              osems[b]).wait()

    in_start(0, 0)

    def outer(o, _):
        for b in range(2):
            i = 2 * o + b
            nxt = 1 - b
            pl.when(i + 1 < BASE_BLOCKS)(lambda: in_start(i + 1, nxt))
            in_wait(i, b)
            pl.when(i >= 2)(lambda: out_wait(i - 2, b))
            _transpose_block(ibuf.at[b], tbuf.at[b], iot, h64v, shalf)
            out_start(i, b)
        return 0

    lax.fori_loop(0, BASE_BLOCKS // 2, outer, 0)
    out_wait(BASE_BLOCKS - 2, 0)
    out_wait(BASE_BLOCKS - 1, 1)

    @pl.when(wid < EXTRA_WORKERS)
    def _extra():
        j = start + BASE_BLOCKS
        pltpu.sync_copy(tt_hbm.at[:, pl.ds(j * 128, 128)], ibuf.at[0])
        _transpose_block(ibuf.at[0], tbuf.at[0], iot, h64v, shalf)
        pltpu.sync_copy(tbuf.at[0], t2s_hbm.at[pl.ds(j * 64, 64)])

    @pl.when(wid == NUM_WORKERS - 1)
    def _tail():
        pltpu.sync_copy(tail_hbm, tailv)
        for s0 in range(0, TAIL_ROWS // 2, 2):
            vs = []
            for s in (s0, s0 + 1):
                for j8 in range(8):
                    rl = 2 * s + (1 if j8 >= 4 else 0)
                    idxv = cvecs[j8 % 4] * 64 + _splat(rl)
                    vs.append(plsc.load_gather(tailv, [idxv]))
            k = 0
            for s in (s0, s0 + 1):
                for j8 in range(8):
                    tbuf[0, s, pl.ds(16 * j8, 16)] = vs[k]
                    k += 1
        pltpu.sync_copy(tbuf.at[0, pl.ds(0, TAIL_ROWS // 2)],
                        t2s_hbm.at[pl.ds(FULL_BLOCKS * 64, TAIL_ROWS // 2)])


def _gather_body(xt_hbm, posf_hbm, t2s_hbm, out_hbm,
                 idx_v, sv, rows_v, otile_v, posf_v,
                 isem0, isem1, rsem0, rsem1, wsem0, wsem1):
    cidx = lax.axis_index("c")
    sidx = lax.axis_index("s")
    wid = sidx * 2 + cidx
    base = wid * UNITS_PER_W

    iot = _iota16()
    isems = (isem0, isem1)
    rsems = (rsem0, rsem1)
    wsems = (wsem0, wsem1)

    pltpu.sync_copy(posf_hbm, posf_v)

    def unit_pb(i):
        uid = base + i
        return uid // 8, uid % 8

    def idx_start(i, b):
        p, bb = unit_pb(i)
        pltpu.async_copy(xt_hbm.at[p, bb], idx_v.at[b], isems[b])

    def idx_wait(i, b):
        p, bb = unit_pb(i)
        pltpu.make_async_copy(xt_hbm.at[p, bb], idx_v.at[b],
                              isems[b]).wait()

    def rows_start(b):
        pltpu.async_copy(t2s_hbm.at[sv.at[b]], rows_v.at[b], rsems[b])

    def rows_wait(b):
        pltpu.make_async_copy(t2s_hbm.at[sv.at[b]], rows_v.at[b],
                              rsems[b]).wait()

    def sv_compute(b):
        for g in range(8):
            sl = pl.ds(g * 16, 16)
            sv[b, sl] = lax.shift_right_logical(idx_v[b, sl], 1)

    def out_start(i, b):
        p, bb = unit_pb(i)
        for cb in range(8):
            pltpu.async_copy(otile_v.at[b, pl.ds(cb * 8, 8)],
                             out_hbm.at[p, cb, bb], wsems[b])

    def out_wait(i, b):
        p, bb = unit_pb(i)
        for cb in range(8):
            pltpu.make_async_copy(otile_v.at[b, pl.ds(cb * 8, 8)],
                                  out_hbm.at[p, cb, bb], wsems[b]).wait()

    def compute(i, b):
        p, _ = unit_pb(i)
        hg64 = []
        rowg = []
        for g in range(8):
            idxg = idx_v[b, pl.ds(g * 16, 16)]
            hg64.append(lax.shift_left(lax.bitwise_and(idxg, 1), 6))
            rowg.append(iot + g * 16)

        def cb_body(cB, _):
            for k in range(16):
                dk = lax.bitwise_and(iot + k, 15)
                cdiag = dk + cB * 16
                pidx = dk * 201 + (cB * (16 * 201) + p)
                posv = plsc.load_gather(posf_v, [pidx])
                for g in range(8):
                    v = plsc.load_gather(rows_v.at[b],
                                         [rowg[g], hg64[g] + cdiag])
                    plsc.store_scatter(otile_v.at[b], [cdiag, rowg[g]],
                                       v + posv)
            return 0

        lax.fori_loop(0, 4, cb_body, 0)

    pltpu.sync_copy(xt_hbm.at[base // 8, base % 8], idx_v.at[0])
    sv_compute(0)
    rows_start(0)

    def _next_gather(i, nxt):
        idx_wait(i + 1, nxt)
        sv_compute(nxt)
        rows_start(nxt)

    def outer(o, _):
        for b in range(2):
            i = 2 * o + b
            nxt = 1 - b
            if b == 0:
                idx_start(i + 1, nxt)
                rows_wait(b)
                _next_gather(i, nxt)
            else:
                pl.when(o < UNITS_PER_W // 2 - 1)(
                    lambda: idx_start(i + 1, nxt))
                rows_wait(b)
                pl.when(o < UNITS_PER_W // 2 - 1)(
                    lambda: _next_gather(i, nxt))
            pl.when(i >= 2)(lambda: out_wait(i - 2, b))
            compute(i, b)
            out_start(i, b)
        return 0

    lax.fori_loop(0, UNITS_PER_W // 2, outer, 0)
    out_wait(UNITS_PER_W - 2, 0)
    out_wait(UNITS_PER_W - 1, 1)


@jax.jit
def _emb_call(x, table, pos_enc):
    mesh = plsc.VectorSubcoreMesh(core_axis_name="c", subcore_axis_name="s")

    tt = table.T
    tail = tt[:, FULL_BLOCKS * 128:].reshape(64 * TAIL_ROWS)
    xt3 = x.astype(jnp.int32).T.reshape(CTX, 8, 128)
    posf = jnp.pad(pos_enc.T, ((0, 0), (0, 1))).reshape(EMBED * (CTX + 1))

    conv = functools.partial(
        pl.kernel,
        mesh=mesh,
        out_type=jax.ShapeDtypeStruct((SROWS, 128), jnp.float32),
        scratch_types=[
            pltpu.VMEM((2, 64, 128), jnp.float32),
            pltpu.VMEM((2, 64, 128), jnp.float32),
            pltpu.VMEM((64 * TAIL_ROWS,), jnp.float32),
        ] + [pltpu.SemaphoreType.DMA] * 4,
        compiler_params=pltpu.CompilerParams(needs_layout_passes=False),
    )(_conv_body)
    t2s = conv(tt, tail)

    gath = functools.partial(
        pl.kernel,
        mesh=mesh,
        out_type=jax.ShapeDtypeStruct((CTX, 8, 8, 8, 128), jnp.float32),
        scratch_types=[
            pltpu.VMEM((2, 128), jnp.int32),
            pltpu.VMEM((2, 128), jnp.int32),
            pltpu.VMEM((2, 128, 128), jnp.float32),
            pltpu.VMEM((2, 64, 128), jnp.float32),
            pltpu.VMEM((EMBED * (CTX + 1),), jnp.float32),
        ] + [pltpu.SemaphoreType.DMA] * 6,
        compiler_params=pltpu.CompilerParams(needs_layout_passes=False),
    )(_gather_body)
    out5 = gath(xt3, posf, t2s)

    return out5.transpose((2, 4, 0, 1, 3)).reshape(BATCH, CTX, EMBED)


def kernel(x, table, pos_enc):
    return _emb_call(x, table, pos_enc)


# speedup vs baseline: 5.1358x; 2.5619x over previous
import functools

import jax
import jax.numpy as jnp
from jax import lax
from jax.experimental import pallas as pl
from jax.experimental.pallas import tpu as pltpu
from jax.experimental.pallas import tpu_sc as plsc

VOCAB = 1000000
EMBED = 64
CTX = 200
BATCH = 1024

NUM_WORKERS = 32
SROWS = VOCAB // 2
FULL_BLOCKS = VOCAB // 128
TAIL_ROWS = VOCAB - FULL_BLOCKS * 128
BASE_BLOCKS = FULL_BLOCKS // NUM_WORKERS
EXTRA_WORKERS = FULL_BLOCKS - BASE_BLOCKS * NUM_WORKERS
UNITS = CTX * (BATCH // 128)
UNITS_PER_W = UNITS // NUM_WORKERS


def _iota16():
    return lax.iota(jnp.int32, 16)


def _splat(v):
    return jnp.full((16,), v, jnp.int32)


def _transpose_block(src, dst, io, h64v, shalf):
    def rb_body(rB, _):
        rlv = io + rB * 16
        srowv = shalf + rB * 8
        for cB in range(4):
            scbase = h64v + cB * 16
            for k0 in range(0, 16, 8):
                vs = []
                for k in range(k0, k0 + 8):
                    dk = lax.bitwise_and(io + k, 15)
                    vs.append((scbase + dk,
                               plsc.load_gather(src, [dk + cB * 16, rlv])))
                for scol, v in vs:
                    plsc.store_scatter(dst, [srowv, scol], v)
        return 0

    lax.fori_loop(0, 8, rb_body, 0)


def _conv_body(tt_hbm, tail_hbm, t2s_hbm,
               ibuf, tbuf, tailv, isem0, isem1, osem0, osem1):
    cidx = lax.axis_index("c")
    sidx = lax.axis_index("s")
    wid = sidx * 2 + cidx
    start = wid * BASE_BLOCKS + jnp.minimum(wid, EXTRA_WORKERS)

    iot = _iota16()
    cvecs = [iot + 16 * k for k in range(4)]
    h64v = lax.shift_left(lax.bitwise_and(iot, 1), 6)
    shalf = lax.shift_right_logical(iot, 1)
    isems = (isem0, isem1)
    osems = (osem0, osem1)

    def in_start(i, b):
        j = start + i
        pltpu.async_copy(tt_hbm.at[:, pl.ds(j * 128, 128)], ibuf.at[b],
                         isems[b])

    def in_wait(i, b):
        j = start + i
        pltpu.make_async_copy(tt_hbm.at[:, pl.ds(j * 128, 128)], ibuf.at[b],
                              isems[b]).wait()

    def out_start(i, b):
        j = start + i
        pltpu.async_copy(tbuf.at[b], t2s_hbm.at[pl.ds(j * 64, 64)], osems[b])

    def out_wait(i, b):
        j = start + i
        pltpu.make_async_copy(tbuf.at[b], t2s_hbm.at[pl.ds(j * 64, 64)],
                              osems[b]).wait()

    in_start(0, 0)

    def outer(o, _):
        for b in range(2):
            i = 2 * o + b
            nxt = 1 - b
            pl.when(i + 1 < BASE_BLOCKS)(lambda: in_start(i + 1, nxt))
            in_wait(i, b)
            pl.when(i >= 2)(lambda: out_wait(i - 2, b))
            _transpose_block(ibuf.at[b], tbuf.at[b], iot, h64v, shalf)
            out_start(i, b)
        return 0

    lax.fori_loop(0, BASE_BLOCKS // 2, outer, 0)
    out_wait(BASE_BLOCKS - 2, 0)
    out_wait(BASE_BLOCKS - 1, 1)

    @pl.when(wid < EXTRA_WORKERS)
    def _extra():
        j = start + BASE_BLOCKS
        pltpu.sync_copy(tt_hbm.at[:, pl.ds(j * 128, 128)], ibuf.at[0])
        _transpose_block(ibuf.at[0], tbuf.at[0], iot, h64v, shalf)
        pltpu.sync_copy(tbuf.at[0], t2s_hbm.at[pl.ds(j * 64, 64)])

    @pl.when(wid == NUM_WORKERS - 1)
    def _tail():
        pltpu.sync_copy(tail_hbm, tailv)
        for s0 in range(0, TAIL_ROWS // 2, 2):
            vs = []
            for s in (s0, s0 + 1):
                for j8 in range(8):
                    rl = 2 * s + (1 if j8 >= 4 else 0)
                    idxv = cvecs[j8 % 4] * 64 + _splat(rl)
                    vs.append(plsc.load_gather(tailv, [idxv]))
            k = 0
            for s in (s0, s0 + 1):
                for j8 in range(8):
                    tbuf[0, s, pl.ds(16 * j8, 16)] = vs[k]
                    k += 1
        pltpu.sync_copy(tbuf.at[0, pl.ds(0, TAIL_ROWS // 2)],
                        t2s_hbm.at[pl.ds(FULL_BLOCKS * 64, TAIL_ROWS // 2)])


def _gather_body(xt_hbm, posf_hbm, t2s_hbm, out_hbm,
                 idx_v, sv, rows_v, otile_v, posf_v,
                 isem0, isem1, rsem0, rsem1, wsem0, wsem1):
    cidx = lax.axis_index("c")
    sidx = lax.axis_index("s")
    wid = sidx * 2 + cidx
    base = wid * UNITS_PER_W

    iot = _iota16()
    isems = (isem0, isem1)
    rsems = (rsem0, rsem1)
    wsems = (wsem0, wsem1)

    pltpu.sync_copy(posf_hbm, posf_v)

    def unit_pb(i):
        uid = base + i
        return uid // 8, uid % 8

    def idx_start(i, b):
        p, bb = unit_pb(i)
        pltpu.async_copy(xt_hbm.at[p, bb], idx_v.at[b], isems[b])

    def idx_wait(i, b):
        p, bb = unit_pb(i)
        pltpu.make_async_copy(xt_hbm.at[p, bb], idx_v.at[b],
                              isems[b]).wait()

    def rows_start(b):
        pltpu.async_copy(t2s_hbm.at[sv.at[b]], rows_v.at[b], rsems[b])

    def rows_wait(b):
        pltpu.make_async_copy(t2s_hbm.at[sv.at[b]], rows_v.at[b],
                              rsems[b]).wait()

    def sv_compute(b):
        for g in range(8):
            sl = pl.ds(g * 16, 16)
            sv[b, sl] = lax.shift_right_logical(idx_v[b, sl], 1)

    def out_start(i, b):
        p, bb = unit_pb(i)
        for cb in range(8):
            pltpu.async_copy(otile_v.at[b, pl.ds(cb * 8, 8)],
                             out_hbm.at[p, cb, bb], wsems[b])

    def out_wait(i, b):
        p, bb = unit_pb(i)
        for cb in range(8):
            pltpu.make_async_copy(otile_v.at[b, pl.ds(cb * 8, 8)],
                                  out_hbm.at[p, cb, bb], wsems[b]).wait()

    def compute(i, b):
        p, _ = unit_pb(i)
        hg64 = []
        rowg = []
        for g in range(8):
            idxg = idx_v[b, pl.ds(g * 16, 16)]
            hg64.append(lax.shift_left(lax.bitwise_and(idxg, 1), 6))
            rowg.append(iot + g * 16)

        def cb_body(cB, _):
            for k in range(16):
                dk = lax.bitwise_and(iot + k, 15)
                cdiag = dk + cB * 16
                pidx = dk * 201 + (cB * (16 * 201) + p)
                posv = plsc.load_gather(posf_v, [pidx])
                vs = [plsc.load_gather(rows_v.at[b],
                                       [rowg[g], hg64[g] + cdiag])
                      for g in range(8)]
                for g in range(8):
                    plsc.store_scatter(otile_v.at[b], [cdiag, rowg[g]],
                                       vs[g] + posv)
            return 0

        lax.fori_loop(0, 4, cb_body, 0)

    pltpu.sync_copy(xt_hbm.at[base // 8, base % 8], idx_v.at[0])
    sv_compute(0)
    rows_start(0)

    def _next_gather(i, nxt):
        idx_wait(i + 1, nxt)
        sv_compute(nxt)
        rows_start(nxt)

    def outer(o, _):
        for b in range(2):
            i = 2 * o + b
            nxt = 1 - b
            if b == 0:
                idx_start(i + 1, nxt)
                rows_wait(b)
                _next_gather(i, nxt)
            else:
                pl.when(o < UNITS_PER_W // 2 - 1)(
                    lambda: idx_start(i + 1, nxt))
                rows_wait(b)
                pl.when(o < UNITS_PER_W // 2 - 1)(
                    lambda: _next_gather(i, nxt))
            pl.when(i >= 2)(lambda: out_wait(i - 2, b))
            compute(i, b)
            out_start(i, b)
        return 0

    lax.fori_loop(0, UNITS_PER_W // 2, outer, 0)
    out_wait(UNITS_PER_W - 2, 0)
    out_wait(UNITS_PER_W - 1, 1)


@jax.jit
def _emb_call(x, table, pos_enc):
    mesh = plsc.VectorSubcoreMesh(core_axis_name="c", subcore_axis_name="s")

    tt = table.T
    tail = tt[:, FULL_BLOCKS * 128:].reshape(64 * TAIL_ROWS)
    xt3 = x.astype(jnp.int32).T.reshape(CTX, 8, 128)
    posf = jnp.pad(pos_enc.T, ((0, 0), (0, 1))).reshape(EMBED * (CTX + 1))

    conv = functools.partial(
        pl.kernel,
        mesh=mesh,
        out_type=jax.ShapeDtypeStruct((SROWS, 128), jnp.float32),
        scratch_types=[
            pltpu.VMEM((2, 64, 128), jnp.float32),
            pltpu.VMEM((2, 64, 128), jnp.float32),
            pltpu.VMEM((64 * TAIL_ROWS,), jnp.float32),
        ] + [pltpu.SemaphoreType.DMA] * 4,
        compiler_params=pltpu.CompilerParams(needs_layout_passes=False),
    )(_conv_body)
    t2s = conv(tt, tail)

    gath = functools.partial(
        pl.kernel,
        mesh=mesh,
        out_type=jax.ShapeDtypeStruct((CTX, 8, 8, 8, 128), jnp.float32),
        scratch_types=[
            pltpu.VMEM((2, 128), jnp.int32),
            pltpu.VMEM((2, 128), jnp.int32),
            pltpu.VMEM((2, 128, 128), jnp.float32),
            pltpu.VMEM((2, 64, 128), jnp.float32),
            pltpu.VMEM((EMBED * (CTX + 1),), jnp.float32),
        ] + [pltpu.SemaphoreType.DMA] * 6,
        compiler_params=pltpu.CompilerParams(needs_layout_passes=False),
    )(_gather_body)
    out5 = gath(xt3, posf, t2s)

    return out5.transpose((2, 4, 0, 1, 3)).reshape(BATCH, CTX, EMBED)


def kernel(x, table, pos_enc):
    return _emb_call(x, table, pos_enc)


# speedup vs baseline: 5.7163x; 1.1130x over previous
import functools

import jax
import jax.numpy as jnp
from jax import lax
from jax.experimental import pallas as pl
from jax.experimental.pallas import tpu as pltpu
from jax.experimental.pallas import tpu_sc as plsc

VOCAB = 1000000
EMBED = 64
CTX = 200
BATCH = 1024

NUM_WORKERS = 32
SROWS = VOCAB // 2
FULL_BLOCKS = VOCAB // 256
TAIL_ROWS = VOCAB - FULL_BLOCKS * 256
BASE_BLOCKS = FULL_BLOCKS // NUM_WORKERS
EXTRA_WORKERS = FULL_BLOCKS - BASE_BLOCKS * NUM_WORKERS
UNITS = CTX * (BATCH // 128)
UNITS_PER_W = UNITS // NUM_WORKERS


def _iota16():
    return lax.iota(jnp.int32, 16)


def _splat(v):
    return jnp.full((16,), v, jnp.int32)


def _transpose_block(src, dst, io, h64v, shalf, dks, n_rb=16):
    def rb_body(rB, _):
        rlv = io + rB * 16
        srowv = shalf + rB * 8
        for cB in range(4):
            scbase = h64v + cB * 16
            for k0 in range(0, 16, 8):
                vs = []
                for k in range(k0, k0 + 8):
                    vs.append((scbase + dks[k],
                               plsc.load_gather(src,
                                                [dks[k] + cB * 16, rlv])))
                for scol, v in vs:
                    plsc.store_scatter(dst, [srowv, scol], v)
        return 0

    lax.fori_loop(0, n_rb, rb_body, 0)


def _conv_body(tt_hbm, tail_hbm, t2s_hbm,
               ibuf, tbuf, tailv, isem0, isem1, osem0, osem1):
    cidx = lax.axis_index("c")
    sidx = lax.axis_index("s")
    wid = sidx * 2 + cidx
    start = wid * BASE_BLOCKS + jnp.minimum(wid, EXTRA_WORKERS)

    iot = _iota16()
    cvecs = [iot + 16 * k for k in range(4)]
    h64v = lax.shift_left(lax.bitwise_and(iot, 1), 6)
    shalf = lax.shift_right_logical(iot, 1)
    dks = [lax.bitwise_and(iot + k, 15) for k in range(16)]
    isems = (isem0, isem1)
    osems = (osem0, osem1)

    def in_start(i, b):
        j = start + i
        pltpu.async_copy(tt_hbm.at[:, pl.ds(j * 256, 256)], ibuf.at[b],
                         isems[b])

    def in_wait(i, b):
        j = start + i
        pltpu.make_async_copy(tt_hbm.at[:, pl.ds(j * 256, 256)], ibuf.at[b],
                              isems[b]).wait()

    def out_start(i, b):
        j = start + i
        pltpu.async_copy(tbuf.at[b], t2s_hbm.at[pl.ds(j * 128, 128)],
                         osems[b])

    def out_wait(i, b):
        j = start + i
        pltpu.make_async_copy(tbuf.at[b], t2s_hbm.at[pl.ds(j * 128, 128)],
                              osems[b]).wait()

    in_start(0, 0)

    def outer(o, _):
        for b in range(2):
            i = 2 * o + b
            nxt = 1 - b
            pl.when(i + 1 < BASE_BLOCKS)(lambda: in_start(i + 1, nxt))
            in_wait(i, b)
            pl.when(i >= 2)(lambda: out_wait(i - 2, b))
            _transpose_block(ibuf.at[b], tbuf.at[b], iot, h64v, shalf, dks)
            out_start(i, b)
        return 0

    lax.fori_loop(0, BASE_BLOCKS // 2, outer, 0)
    out_wait(BASE_BLOCKS - 2, 0)
    out_wait(BASE_BLOCKS - 1, 1)

    @pl.when(wid < EXTRA_WORKERS)
    def _extra():
        j = start + BASE_BLOCKS
        pltpu.sync_copy(tt_hbm.at[:, pl.ds(j * 256, 256)], ibuf.at[0])
        _transpose_block(ibuf.at[0], tbuf.at[0], iot, h64v, shalf, dks)
        pltpu.sync_copy(tbuf.at[0], t2s_hbm.at[pl.ds(j * 128, 128)])

    @pl.when(wid == NUM_WORKERS - 1)
    def _tail():
        pltpu.sync_copy(tail_hbm, tailv)
        for s0 in range(0, TAIL_ROWS // 2, 2):
            vs = []
            for s in (s0, s0 + 1):
                for j8 in range(8):
                    rl = 2 * s + (1 if j8 >= 4 else 0)
                    idxv = cvecs[j8 % 4] * 64 + _splat(rl)
                    vs.append(plsc.load_gather(tailv, [idxv]))
            k = 0
            for s in (s0, s0 + 1):
                for j8 in range(8):
                    tbuf[0, s, pl.ds(16 * j8, 16)] = vs[k]
                    k += 1
        pltpu.sync_copy(tbuf.at[0, pl.ds(0, TAIL_ROWS // 2)],
                        t2s_hbm.at[pl.ds(FULL_BLOCKS * 128, TAIL_ROWS // 2)])


def _gather_body(xt_hbm, posf_hbm, t2s_hbm, out_hbm,
                 idx_v, sv, rows_v, otile_v, posf_v,
                 isem0, isem1, rsem0, rsem1, wsem0, wsem1):
    cidx = lax.axis_index("c")
    sidx = lax.axis_index("s")
    wid = sidx * 2 + cidx
    base = wid * UNITS_PER_W

    iot = _iota16()
    isems = (isem0, isem1)
    rsems = (rsem0, rsem1)
    wsems = (wsem0, wsem1)

    pltpu.sync_copy(posf_hbm, posf_v)

    def unit_pb(i):
        uid = base + i
        return uid // 8, uid % 8

    def idx_start(i, b):
        p, bb = unit_pb(i)
        pltpu.async_copy(xt_hbm.at[p, bb], idx_v.at[b], isems[b])

    def idx_wait(i, b):
        p, bb = unit_pb(i)
        pltpu.make_async_copy(xt_hbm.at[p, bb], idx_v.at[b],
                              isems[b]).wait()

    def rows_start(b):
        pltpu.async_copy(t2s_hbm.at[sv.at[b]], rows_v.at[b], rsems[b])

    def rows_wait(b):
        pltpu.make_async_copy(t2s_hbm.at[sv.at[b]], rows_v.at[b],
                              rsems[b]).wait()

    def sv_compute(b):
        for g in range(8):
            sl = pl.ds(g * 16, 16)
            sv[b, sl] = lax.shift_right_logical(idx_v[b, sl], 1)

    def out_start(i, b):
        p, bb = unit_pb(i)
        for cb in range(8):
            pltpu.async_copy(otile_v.at[b, pl.ds(cb * 8, 8)],
                             out_hbm.at[p, cb, bb], wsems[b])

    def out_wait(i, b):
        p, bb = unit_pb(i)
        for cb in range(8):
            pltpu.make_async_copy(otile_v.at[b, pl.ds(cb * 8, 8)],
                                  out_hbm.at[p, cb, bb], wsems[b]).wait()

    def compute(i, b):
        p, _ = unit_pb(i)
        hg64 = []
        rowg = []
        for g in range(8):
            idxg = idx_v[b, pl.ds(g * 16, 16)]
            hg64.append(lax.shift_left(lax.bitwise_and(idxg, 1), 6))
            rowg.append(iot + g * 16)

        def cb_body(cB, _):
            for k in range(16):
                dk = lax.bitwise_and(iot + k, 15)
                cdiag = dk + cB * 16
                pidx = dk * 201 + (cB * (16 * 201) + p)
                posv = plsc.load_gather(posf_v, [pidx])
                vs = [plsc.load_gather(rows_v.at[b],
                                       [rowg[g], hg64[g] + cdiag])
                      for g in range(8)]
                for g in range(8):
                    plsc.store_scatter(otile_v.at[b], [cdiag, rowg[g]],
                                       vs[g] + posv)
            return 0

        lax.fori_loop(0, 4, cb_body, 0)

    pltpu.sync_copy(xt_hbm.at[base // 8, base % 8], idx_v.at[0])
    sv_compute(0)
    rows_start(0)

    def _next_gather(i, nxt):
        idx_wait(i + 1, nxt)
        sv_compute(nxt)
        rows_start(nxt)

    def outer(o, _):
        for b in range(2):
            i = 2 * o + b
            nxt = 1 - b
            if b == 0:
                idx_start(i + 1, nxt)
                rows_wait(b)
                _next_gather(i, nxt)
            else:
                pl.when(o < UNITS_PER_W // 2 - 1)(
                    lambda: idx_start(i + 1, nxt))
                rows_wait(b)
                pl.when(o < UNITS_PER_W // 2 - 1)(
                    lambda: _next_gather(i, nxt))
            pl.when(i >= 2)(lambda: out_wait(i - 2, b))
            compute(i, b)
            out_start(i, b)
        return 0

    lax.fori_loop(0, UNITS_PER_W // 2, outer, 0)
    out_wait(UNITS_PER_W - 2, 0)
    out_wait(UNITS_PER_W - 1, 1)


@jax.jit
def _emb_call(x, table, pos_enc):
    mesh = plsc.VectorSubcoreMesh(core_axis_name="c", subcore_axis_name="s")

    tt = table.T
    tail = tt[:, FULL_BLOCKS * 256:].reshape(64 * TAIL_ROWS)
    xt3 = x.astype(jnp.int32).T.reshape(CTX, 8, 128)
    posf = jnp.pad(pos_enc.T, ((0, 0), (0, 1))).reshape(EMBED * (CTX + 1))

    conv = functools.partial(
        pl.kernel,
        mesh=mesh,
        out_type=jax.ShapeDtypeStruct((SROWS, 128), jnp.float32),
        scratch_types=[
            pltpu.VMEM((2, 64, 256), jnp.float32),
            pltpu.VMEM((2, 128, 128), jnp.float32),
            pltpu.VMEM((64 * TAIL_ROWS,), jnp.float32),
        ] + [pltpu.SemaphoreType.DMA] * 4,
        compiler_params=pltpu.CompilerParams(needs_layout_passes=False),
    )(_conv_body)
    t2s = conv(tt, tail)

    gath = functools.partial(
        pl.kernel,
        mesh=mesh,
        out_type=jax.ShapeDtypeStruct((CTX, 8, 8, 8, 128), jnp.float32),
        scratch_types=[
            pltpu.VMEM((2, 128), jnp.int32),
            pltpu.VMEM((2, 128), jnp.int32),
            pltpu.VMEM((2, 128, 128), jnp.float32),
            pltpu.VMEM((2, 64, 128), jnp.float32),
            pltpu.VMEM((EMBED * (CTX + 1),), jnp.float32),
        ] + [pltpu.SemaphoreType.DMA] * 6,
        compiler_params=pltpu.CompilerParams(needs_layout_passes=False),
    )(_gather_body)
    out5 = gath(xt3, posf, t2s)

    return out5.transpose((2, 4, 0, 1, 3)).reshape(BATCH, CTX, EMBED)


def kernel(x, table, pos_enc):
    return _emb_call(x, table, pos_enc)
